# Initial kernel scaffold; baseline (speedup 1.0000x reference)
import jax
import jax.numpy as jnp
from jax.experimental import pallas as pl


def kernel(x, edge_index, target_node_idx, apart_feature, W1l, b1l, W1r, W2l, b2l, W2r, Wq, bq, Wk, bk, Wv, bv, Wf1, bf1, Wf2, bf2, Wf3, bf3):
    raise NotImplementedError("write your pallas kernel here")


# speedup vs baseline: 5.3624x; 5.3624x over previous
import functools
import math

import jax
import jax.numpy as jnp
from jax import lax
from jax.experimental import pallas as pl
from jax.experimental.pallas import tpu as pltpu
from jax.experimental.pallas import tpu_sc as plsc

N = 10000
E = 320000
T = 12
IN_DIM = 12
H = 128
B = 2048
AF = 10

NC = 2
NS = 16
NW = NC * NS
K = 80

FIRE1 = 2
G1 = FIRE1 * K
GPW1 = -(-E // (NW * G1))
E1 = NW * GPW1 * G1

FIRE2 = 4
G2 = FIRE2 * K
GPW2 = -(-E // (NW * G2))
E2 = NW * GPW2 * G2

NP = N + 16
SPLIT = 624
TAIL = NP - SPLIT * NS

F1 = T * IN_DIM
F1A = F1 + 16


@functools.cache
def _mesh():
    return plsc.VectorSubcoreMesh(core_axis_name="c", subcore_axis_name="s",
                                  num_cores=NC, num_subcores=NS)


def _wid():
    return lax.axis_index("s") * NC + lax.axis_index("c")


def _striped_copy(src, dst, s_base, d_base, sid):
    o1 = pl.multiple_of(s_base + sid * SPLIT, 8)
    o2 = pl.multiple_of(d_base + sid * SPLIT, 8)
    pltpu.sync_copy(src.at[pl.ds(o1, SPLIT)], dst.at[pl.ds(o2, SPLIT)])

    @pl.when(sid == NS - 1)
    def _():
        t1 = pl.multiple_of(s_base + SPLIT * NS, 8)
        t2 = pl.multiple_of(d_base + SPLIT * NS, 8)
        pltpu.sync_copy(src.at[pl.ds(t1, TAIL)], dst.at[pl.ds(t2, TAIL)])


@functools.cache
def _sc_agg1_kernel():
    return pl.kernel(
        _sc_agg1_body,
        out_type=jax.ShapeDtypeStruct((NC * NP, F1A), jnp.float32),
        mesh=_mesh(),
        scratch_types=[
            pltpu.VMEM_SHARED((NP, F1A), jnp.float32),
            pltpu.VMEM((FIRE1, K), jnp.int32),
            pltpu.VMEM((FIRE1, K), jnp.int32),
            pltpu.VMEM((FIRE1 * K, F1A), jnp.float32),
            pltpu.SemaphoreType.DMA,
            pltpu.SemaphoreType.DMA,
        ],
        compiler_params=pltpu.CompilerParams(use_tc_tiling_on_sc=False),
    )


def _sc_agg1_body(x_hbm, src_hbm, dst_hbm, z1_hbm, agg_hbm,
                  slab, src_v, dst_v, rows, sem_g, sem_s):
    cid = lax.axis_index("c")
    sid = lax.axis_index("s")
    wid = _wid()

    _striped_copy(z1_hbm, slab, 0, 0, sid)
    plsc.subcore_barrier()

    @pl.loop(0, GPW1)
    def _(g):
        wsc = wid * GPW1 + g
        pltpu.sync_copy(src_hbm.at[wsc], src_v)
        pltpu.sync_copy(dst_hbm.at[wsc], dst_v)
        gets = [pltpu.async_copy(x_hbm.at[src_v.at[j]],
                                 rows.at[pl.ds(j * K, K)], sem_g)
                for j in range(FIRE1)]
        for d in gets:
            d.wait()
        puts = [pltpu.async_copy(rows.at[pl.ds(j * K, K)],
                                 slab.at[dst_v.at[j]], sem_s, add=True)
                for j in range(FIRE1)]
        for d in puts:
            d.wait()

    plsc.subcore_barrier()
    _striped_copy(slab, agg_hbm, 0, cid * NP, sid)


@functools.cache
def _sc_agg2_kernel():
    return pl.kernel(
        _sc_agg2_body,
        out_type=jax.ShapeDtypeStruct((NC * T * NP, H), jnp.float32),
        mesh=_mesh(),
        scratch_types=[
            pltpu.VMEM_SHARED((NP, H), jnp.float32),
            pltpu.VMEM((FIRE2, K), jnp.int32),
            pltpu.VMEM((FIRE2, K), jnp.int32),
            pltpu.VMEM((FIRE2 * K, H), jnp.float32),
            pltpu.SemaphoreType.DMA,
            pltpu.SemaphoreType.DMA,
        ],
        compiler_params=pltpu.CompilerParams(use_tc_tiling_on_sc=False),
    )


def _sc_agg2_body(r0, r1, r2, r3, r4, r5, r6, r7, r8, r9, r10, r11,
                  src_hbm, dst_hbm, z2_hbm, agg_hbm,
                  slab, src_v, dst_v, rows, sem_g, sem_s):
    cid = lax.axis_index("c")
    sid = lax.axis_index("s")
    wid = _wid()
    regs = [r0, r1, r2, r3, r4, r5, r6, r7, r8, r9, r10, r11]

    for t in range(T):
        _striped_copy(z2_hbm, slab, 0, 0, sid)
        plsc.subcore_barrier()

        @pl.loop(0, GPW2)
        def _(g):
            wsc = wid * GPW2 + g
            pltpu.sync_copy(src_hbm.at[wsc], src_v)
            pltpu.sync_copy(dst_hbm.at[wsc], dst_v)
            gets = [pltpu.async_copy(regs[t].at[src_v.at[j]],
                                     rows.at[pl.ds(j * K, K)], sem_g)
                    for j in range(FIRE2)]
            for d in gets:
                d.wait()
            puts = [pltpu.async_copy(rows.at[pl.ds(j * K, K)],
                                     slab.at[dst_v.at[j]], sem_s, add=True)
                    for j in range(FIRE2)]
            for d in puts:
                d.wait()

        plsc.subcore_barrier()
        _striped_copy(slab, agg_hbm, 0, cid * T * NP + t * NP, sid)
        plsc.subcore_barrier()


@functools.cache
def _sc_gather_targets_kernel():
    return pl.kernel(
        _sc_gather_targets_body,
        out_type=jax.ShapeDtypeStruct((B, H), jnp.float32),
        mesh=_mesh(),
        scratch_types=[
            pltpu.VMEM((B // NW,), jnp.int32),
            pltpu.VMEM((B // NW, H), jnp.float32),
            pltpu.SemaphoreType.DMA,
        ],
        compiler_params=pltpu.CompilerParams(use_tc_tiling_on_sc=False),
    )


def _sc_gather_targets_body(feat_hbm, tgt_hbm, out_hbm, idx_v, rows, sem):
    wid = _wid()
    bpw = B // NW
    pltpu.sync_copy(tgt_hbm.at[pl.ds(wid * bpw, bpw)], idx_v)
    pltpu.async_copy(feat_hbm.at[idx_v], rows, sem).wait()
    pltpu.sync_copy(rows, out_hbm.at[pl.ds(wid * bpw, bpw)])


NB = 400
ISQ = 1.0 / math.sqrt(H)


def _bf(x):
    return x.astype(jnp.bfloat16).astype(jnp.float32)


def _attn_block(hs, wqkv_ref, bq_ref, bk_ref, bv_ref):
    qs, ks, vs = [], [], []
    for t in range(T):
        qkv = jnp.dot(hs[t], wqkv_ref[...], preferred_element_type=jnp.float32)
        qs.append(_bf((qkv[:, :H] + bq_ref[...]) * ISQ))
        ks.append(_bf(qkv[:, H:2 * H] + bk_ref[...]))
        vs.append(_bf(qkv[:, 2 * H:] + bv_ref[...]))
    outs = []
    for i in range(T):
        s = jnp.concatenate(
            [jnp.sum(qs[i] * ks[j], axis=-1, keepdims=True) for j in range(T)],
            axis=-1)
        w = _bf(jax.nn.softmax(s, axis=-1))
        o = w[:, 0:1] * vs[0]
        for j in range(1, T):
            o = o + w[:, j:j + 1] * vs[j]
        outs.append(o)
    return outs


def _tc_layer1_body(x_ref, agg_ref, w1_ref, b1_ref,
                    wqkv_ref, bq_ref, bk_ref, bv_ref, reg_ref):
    deg = jnp.maximum(agg_ref[0, :, F1:F1 + 1] + agg_ref[1, :, F1:F1 + 1], 1.0)
    hs = []
    for t in range(T):
        sl = pl.ds(t * IN_DIM, IN_DIM)
        mean_t = (agg_ref[0, :, sl] + agg_ref[1, :, sl]) / deg
        xt = x_ref[:, sl]
        cat = jnp.concatenate([mean_t, xt], axis=-1)
        h = jnp.dot(cat, w1_ref[...], preferred_element_type=jnp.float32)
        hs.append(jnp.maximum(h + b1_ref[...], 0.0))
    outs = _attn_block(hs, wqkv_ref, bq_ref, bk_ref, bv_ref)
    for t in range(T):
        reg_ref[t, :, :] = outs[t]


def _tc_layer1(x2d, agg1p, w1, b1, wqkv, bq, bk, bv, interpret=False):
    grid = (N // NB,)
    return pl.pallas_call(
        _tc_layer1_body,
        grid=grid,
        in_specs=[
            pl.BlockSpec((NB, F1), lambda i: (i, 0)),
            pl.BlockSpec((2, NB, F1A), lambda i: (0, i, 0)),
            pl.BlockSpec((2 * IN_DIM, H), lambda i: (0, 0)),
            pl.BlockSpec((1, H), lambda i: (0, 0)),
            pl.BlockSpec((H, 3 * H), lambda i: (0, 0)),
            pl.BlockSpec((1, H), lambda i: (0, 0)),
            pl.BlockSpec((1, H), lambda i: (0, 0)),
            pl.BlockSpec((1, H), lambda i: (0, 0)),
        ],
        out_specs=pl.BlockSpec((T, NB, H), lambda i: (0, i, 0)),
        out_shape=jax.ShapeDtypeStruct((T, N, H), jnp.float32),
        interpret=interpret,
    )(x2d, agg1p, w1, b1, wqkv, bq, bk, bv)


def _tc_layer2_body(reg_ref, agg_ref, deg_ref, w2_ref, b2_ref,
                    wqkv_ref, bq_ref, bk_ref, bv_ref, out_ref):
    deg = jnp.maximum(deg_ref[0, :, F1:F1 + 1] + deg_ref[1, :, F1:F1 + 1], 1.0)
    hs = []
    for t in range(T):
        mean_t = (agg_ref[0, t] + agg_ref[1, t]) / deg
        cat = jnp.concatenate([mean_t, reg_ref[t]], axis=-1)
        h = jnp.dot(cat, w2_ref[...], preferred_element_type=jnp.float32)
        hs.append(jnp.maximum(h + b2_ref[...], 0.0))
    outs = _attn_block(hs, wqkv_ref, bq_ref, bk_ref, bv_ref)
    acc = outs[0]
    for t in range(1, T):
        acc = acc + outs[t]
    out_ref[...] = acc


def _tc_layer2(reg, agg2p, agg1p, w2, b2, wqkv, bq, bk, bv, interpret=False):
    grid = (N // NB,)
    return pl.pallas_call(
        _tc_layer2_body,
        grid=grid,
        in_specs=[
            pl.BlockSpec((T, NB, H), lambda i: (0, i, 0)),
            pl.BlockSpec((2, T, NB, H), lambda i: (0, 0, i, 0)),
            pl.BlockSpec((2, NB, F1A), lambda i: (0, i, 0)),
            pl.BlockSpec((2 * H, H), lambda i: (0, 0)),
            pl.BlockSpec((1, H), lambda i: (0, 0)),
            pl.BlockSpec((H, 3 * H), lambda i: (0, 0)),
            pl.BlockSpec((1, H), lambda i: (0, 0)),
            pl.BlockSpec((1, H), lambda i: (0, 0)),
            pl.BlockSpec((1, H), lambda i: (0, 0)),
        ],
        out_specs=pl.BlockSpec((NB, H), lambda i: (i, 0)),
        out_shape=jax.ShapeDtypeStruct((N, H), jnp.float32),
        interpret=interpret,
    )(reg, agg2p, agg1p, w2, b2, wqkv, bq, bk, bv)


def _tc_mlp_body(g_ref, a_ref, wf1_ref, bf1_ref, wf2_ref, bf2_ref,
                 wf3_ref, bf3_ref, out_ref):
    feat = jnp.concatenate([g_ref[...], a_ref[...]], axis=-1)
    h1 = jnp.dot(feat, wf1_ref[...], preferred_element_type=jnp.float32)
    h1 = h1 + bf1_ref[...]
    h1 = jnp.where(h1 > 0, h1, 0.1 * h1)
    h2 = jnp.dot(h1, wf2_ref[...], preferred_element_type=jnp.float32)
    h2 = h2 + bf2_ref[...]
    h2 = jnp.where(h2 > 0, h2, 0.05 * h2)
    h3 = jnp.dot(h2, wf3_ref[...], preferred_element_type=jnp.float32)
    out_ref[...] = h3 + bf3_ref[...]


def _tc_mlp(g, a, wf1, bf1, wf2, bf2, wf3, bf3, interpret=False):
    return pl.pallas_call(
        _tc_mlp_body,
        out_shape=jax.ShapeDtypeStruct((B, 1), jnp.float32),
        interpret=interpret,
    )(g, a, wf1, bf1, wf2, bf2, wf3, bf3)


def _pad_edges(src, dst, etot):
    npad = etot - E
    fill = jnp.arange(npad, dtype=jnp.int32) % 16
    srcp = jnp.concatenate([src, fill])
    dstp = jnp.concatenate([dst, N + fill])
    return srcp, dstp


def kernel(x, edge_index, target_node_idx, apart_feature,
           W1l, b1l, W1r, W2l, b2l, W2r,
           Wq, bq, Wk, bk, Wv, bv,
           Wf1, bf1, Wf2, bf2, Wf3, bf3):
    x2d = jnp.transpose(x, (1, 0, 2)).reshape(N, F1)
    x2da = jnp.concatenate([x2d, jnp.ones((N, 16), jnp.float32)], axis=1)
    src = edge_index[0].astype(jnp.int32)
    dst = edge_index[1].astype(jnp.int32)
    s1, d1 = _pad_edges(src, dst, E1)
    src1 = s1.reshape(NW * GPW1, FIRE1, K)
    dst1 = d1.reshape(NW * GPW1, FIRE1, K)
    s2, d2 = _pad_edges(src, dst, E2)
    src2 = s2.reshape(NW * GPW2, FIRE2, K)
    dst2 = d2.reshape(NW * GPW2, FIRE2, K)
    tgt = target_node_idx.astype(jnp.int32)
    z1 = jnp.zeros((NP, F1A), jnp.float32)
    z2 = jnp.zeros((NP, H), jnp.float32)

    w1 = jnp.concatenate([W1l.T, W1r.T], axis=0)
    w2 = jnp.concatenate([W2l.T, W2r.T], axis=0)
    wqkv = jnp.concatenate([Wq.T, Wk.T, Wv.T], axis=1)
    b1 = b1l.reshape(1, H)
    b2 = b2l.reshape(1, H)
    bq2 = bq.reshape(1, H)
    bk2 = bk.reshape(1, H)
    bv2 = bv.reshape(1, H)
    wf1 = Wf1.T
    wf2 = Wf2.T
    wf3 = Wf3.T
    bf1r = bf1.reshape(1, -1)
    bf2r = bf2.reshape(1, -1)
    bf3r = bf3.reshape(1, -1)

    agg1_flat = _sc_agg1_kernel()(x2da, src1, dst1, z1)
    agg1p = agg1_flat.reshape(2, NP, F1A)[:, :N, :]

    reg = _tc_layer1(x2d, agg1p, w1, b1, wqkv, bq2, bk2, bv2)

    agg2_flat = _sc_agg2_kernel()(*[reg[t] for t in range(T)], src2, dst2, z2)
    agg2p = agg2_flat.reshape(2, T, NP, H)[:, :, :N, :]

    feat = _tc_layer2(reg, agg2p, agg1p, w2, b2, wqkv, bq2, bk2, bv2)

    g = _sc_gather_targets_kernel()(feat, tgt)
    return _tc_mlp(g, apart_feature, wf1, bf1r, wf2, bf2r, wf3, bf3r)
